# Initial kernel scaffold; baseline (speedup 1.0000x reference)
import functools

import jax
import jax.numpy as jnp
from jax import lax
from jax.experimental import pallas as pl
from jax.experimental.pallas import tpu as pltpu
from jax.experimental.pallas import tpu_sc as plsc

E = 8
K = 2
D = 1024
FF = 4096
T = 4096
A = T * K

BLK = 256
P = A + E * BLK
NB = P // BLK

NC, NS = 2, 16
NW = NC * NS


def _router_body(x_ref, gw_ref, gb_ref, a1_ref, a2_ref, p1_ref, p2_ref):
    logits = jnp.dot(x_ref[...], gw_ref[...],
                     preferred_element_type=jnp.float32) + gb_ref[...]
    e_iota = lax.broadcasted_iota(jnp.int32, logits.shape, 1)
    m1 = jnp.max(logits, axis=1, keepdims=True)
    a1 = jnp.min(jnp.where(logits == m1, e_iota, E), axis=1, keepdims=True)
    masked = jnp.where(e_iota == a1, -jnp.inf, logits)
    m2 = jnp.max(masked, axis=1, keepdims=True)
    a2 = jnp.min(jnp.where(masked == m2, e_iota, E), axis=1, keepdims=True)
    p1 = 1.0 / (1.0 + jnp.exp(m2 - m1))
    p2 = 1.0 / (1.0 + jnp.exp(m1 - m2))
    a1_ref[...] = a1
    a2_ref[...] = a2
    p1_ref[...] = p1
    p2_ref[...] = p2


def _router(x, gate_w, gate_b):
    shp = jax.ShapeDtypeStruct((T, 1), jnp.int32)
    shpf = jax.ShapeDtypeStruct((T, 1), jnp.float32)
    return pl.pallas_call(
        _router_body,
        out_shape=(shp, shp, shpf, shpf),
    )(x, gate_w, gate_b.reshape(1, E))


def _route_metadata(a1, a2):
    fe = jnp.concatenate([a1, a2], axis=0).reshape(-1)
    onehot = (fe[:, None] == jnp.arange(E)[None, :]).astype(jnp.int32)
    rank = jnp.cumsum(onehot, axis=0) - onehot
    rank_i = jnp.sum(rank * onehot, axis=1)
    counts = jnp.sum(onehot, axis=0)
    padded = ((counts + BLK - 1) // BLK) * BLK
    seg_end = jnp.cumsum(padded)
    seg_start = seg_end - padded
    dest = seg_start[fe] + rank_i
    src_token = jnp.zeros((P,), jnp.int32).at[dest].set(
        jnp.arange(A, dtype=jnp.int32) % T)
    blk_rows = jnp.arange(NB, dtype=jnp.int32) * BLK
    block_expert = jnp.sum(
        (seg_end[None, :] <= blk_rows[:, None]).astype(jnp.int32), axis=1)
    block_expert = jnp.minimum(block_expert, E - 1)
    d = dest.reshape(K, T)
    return src_token, block_expert, d[0], d[1]


_SC_MESH = plsc.VectorSubcoreMesh(core_axis_name="c", subcore_axis_name="s")

_G_CH = 32
_G_PER_W = P // NW


@functools.partial(
    pl.kernel,
    out_type=jax.ShapeDtypeStruct((P, D), jnp.float32),
    mesh=_SC_MESH,
    scratch_types=[
        pltpu.VMEM((_G_CH,), jnp.int32),
        pltpu.VMEM((_G_CH, D), jnp.float32),
        pltpu.SemaphoreType.DMA,
    ],
)
def _dispatch(x_hbm, idx_hbm, out_hbm, idx_v, rows_v, sem):
    wid = lax.axis_index("s") * NC + lax.axis_index("c")
    base = wid * _G_PER_W

    def chunk(i, carry):
        off = base + i * _G_CH
        pltpu.sync_copy(idx_hbm.at[pl.ds(off, _G_CH)], idx_v)
        pltpu.async_copy(x_hbm.at[idx_v], rows_v, sem).wait()
        pltpu.sync_copy(rows_v, out_hbm.at[pl.ds(off, _G_CH)])
        return carry

    lax.fori_loop(0, _G_PER_W // _G_CH, chunk, 0)


def _ffn_body(be_ref, x_ref, w1_ref, b1_ref, w2_ref, b2_ref, o_ref):
    xb = x_ref[...].astype(jnp.bfloat16)
    h = jnp.dot(xb, w1_ref[0], preferred_element_type=jnp.float32)
    h = h + b1_ref[...]
    h = h / (1.0 + jnp.exp(-h))
    y = jnp.dot(h.astype(jnp.bfloat16), w2_ref[0],
                preferred_element_type=jnp.float32)
    o_ref[...] = y + b2_ref[...]


def _grouped_ffn(block_expert, x_sorted, w1, b1, w2, b2):
    grid_spec = pltpu.PrefetchScalarGridSpec(
        num_scalar_prefetch=1,
        grid=(NB,),
        in_specs=[
            pl.BlockSpec((BLK, D), lambda b, be: (b, 0)),
            pl.BlockSpec((1, D, FF), lambda b, be: (be[b], 0, 0)),
            pl.BlockSpec((1, FF), lambda b, be: (be[b], 0)),
            pl.BlockSpec((1, FF, D), lambda b, be: (be[b], 0, 0)),
            pl.BlockSpec((1, D), lambda b, be: (be[b], 0)),
        ],
        out_specs=pl.BlockSpec((BLK, D), lambda b, be: (b, 0)),
    )
    return pl.pallas_call(
        _ffn_body,
        grid_spec=grid_spec,
        out_shape=jax.ShapeDtypeStruct((P, D), jnp.float32),
    )(block_expert, x_sorted, w1, b1, w2, b2)


_C_CH = 16
_C_PER_W = T // NW


@functools.partial(
    pl.kernel,
    out_type=jax.ShapeDtypeStruct((T, D), jnp.float32),
    mesh=_SC_MESH,
    scratch_types=[
        pltpu.VMEM((_C_CH,), jnp.int32),
        pltpu.VMEM((_C_CH,), jnp.int32),
        pltpu.VMEM((_C_CH,), jnp.float32),
        pltpu.VMEM((_C_CH,), jnp.float32),
        pltpu.VMEM((_C_CH, D), jnp.float32),
        pltpu.VMEM((_C_CH, D), jnp.float32),
        pltpu.VMEM((_C_CH, D), jnp.float32),
        pltpu.SemaphoreType.DMA,
        pltpu.SemaphoreType.DMA,
    ],
)
def _combine(y_hbm, d0_hbm, d1_hbm, p0_hbm, p1_hbm, out_hbm,
             i0_v, i1_v, q0_v, q1_v, r0_v, r1_v, o_v, sem0, sem1):
    wid = lax.axis_index("s") * NC + lax.axis_index("c")
    base = wid * _C_PER_W

    def chunk(i, carry):
        off = base + i * _C_CH
        pltpu.sync_copy(d0_hbm.at[pl.ds(off, _C_CH)], i0_v)
        pltpu.sync_copy(d1_hbm.at[pl.ds(off, _C_CH)], i1_v)
        pltpu.sync_copy(p0_hbm.at[pl.ds(off, _C_CH)], q0_v)
        pltpu.sync_copy(p1_hbm.at[pl.ds(off, _C_CH)], q1_v)
        c0 = pltpu.async_copy(y_hbm.at[i0_v], r0_v, sem0)
        c1 = pltpu.async_copy(y_hbm.at[i1_v], r1_v, sem1)
        c0.wait()
        c1.wait()
        for r in range(_C_CH):
            rp = jnp.full((16,), r, jnp.int32)
            s0 = plsc.load_gather(q0_v, [rp])
            s1 = plsc.load_gather(q1_v, [rp])

            def col(c, carry2):
                sl = pl.ds(c * 16, 16)
                o_v[r, sl] = r0_v[r, sl] * s0 + r1_v[r, sl] * s1
                return carry2

            lax.fori_loop(0, D // 16, col, 0)
        pltpu.sync_copy(o_v, out_hbm.at[pl.ds(off, _C_CH)])
        return carry

    lax.fori_loop(0, _C_PER_W // _C_CH, chunk, 0)


def kernel(inputs, gate_w, gate_b, w1, b1, w2, b2):
    a1, a2, p1, p2 = _router(inputs, gate_w, gate_b)
    src_token, block_expert, d0, d1 = _route_metadata(a1, a2)
    x_sorted = _dispatch(inputs, src_token)
    y_sorted = _grouped_ffn(block_expert, x_sorted,
                            w1.astype(jnp.bfloat16), b1,
                            w2.astype(jnp.bfloat16), b2)
    return _combine(y_sorted, d0, d1, p1.reshape(-1), p2.reshape(-1))


# speedup vs baseline: 1.3323x; 1.3323x over previous
import functools

import jax
import jax.numpy as jnp
from jax import lax
from jax.experimental import pallas as pl
from jax.experimental.pallas import tpu as pltpu
from jax.experimental.pallas import tpu_sc as plsc

E = 8
K = 2
D = 1024
FF = 4096
T = 4096
A = T * K

BLK = 256
P = A + E * BLK
NB = P // BLK

NC, NS = 2, 16
NW = NC * NS


def _router_body(x_ref, gw_ref, gb_ref, a1_ref, a2_ref, p1_ref, p2_ref):
    logits = jnp.dot(x_ref[...], gw_ref[...],
                     preferred_element_type=jnp.float32) + gb_ref[...]
    e_iota = lax.broadcasted_iota(jnp.int32, logits.shape, 1)
    m1 = jnp.max(logits, axis=1, keepdims=True)
    a1 = jnp.min(jnp.where(logits == m1, e_iota, E), axis=1, keepdims=True)
    masked = jnp.where(e_iota == a1, -jnp.inf, logits)
    m2 = jnp.max(masked, axis=1, keepdims=True)
    a2 = jnp.min(jnp.where(masked == m2, e_iota, E), axis=1, keepdims=True)
    p1 = 1.0 / (1.0 + jnp.exp(m2 - m1))
    p2 = 1.0 / (1.0 + jnp.exp(m1 - m2))
    a1_ref[...] = a1
    a2_ref[...] = a2
    p1_ref[...] = p1
    p2_ref[...] = p2


def _router(x, gate_w, gate_b):
    shp = jax.ShapeDtypeStruct((T, 1), jnp.int32)
    shpf = jax.ShapeDtypeStruct((T, 1), jnp.float32)
    return pl.pallas_call(
        _router_body,
        out_shape=(shp, shp, shpf, shpf),
    )(x, gate_w, gate_b.reshape(1, E))


def _route_metadata(a1, a2, p1, p2):
    fe = jnp.concatenate([a1, a2], axis=0).reshape(-1)
    onehot = (fe[:, None] == jnp.arange(E)[None, :]).astype(jnp.int32)
    rank = jnp.cumsum(onehot, axis=0) - onehot
    rank_i = jnp.sum(rank * onehot, axis=1)
    counts = jnp.sum(onehot, axis=0)
    padded = ((counts + BLK - 1) // BLK) * BLK
    seg_end = jnp.cumsum(padded)
    seg_start = seg_end - padded
    dest = seg_start[fe] + rank_i
    src_token = jnp.zeros((P,), jnp.int32).at[dest].set(
        jnp.arange(A, dtype=jnp.int32) % T)
    wflat = jnp.concatenate([p1, p2], axis=0).reshape(-1)
    w_sorted = jnp.zeros((P,), jnp.float32).at[dest].set(wflat)
    blk_rows = jnp.arange(NB, dtype=jnp.int32) * BLK
    block_expert = jnp.sum(
        (seg_end[None, :] <= blk_rows[:, None]).astype(jnp.int32), axis=1)
    block_expert = jnp.minimum(block_expert, E - 1)
    d = dest.reshape(K, T)
    return src_token, w_sorted.reshape(P, 1), block_expert, d[0], d[1]


_G_CH = 32
_G_PER_W = P // NW


@functools.cache
def _make_dispatch():
    return functools.partial(
        pl.kernel,
        out_type=jax.ShapeDtypeStruct((P, D), jnp.float32),
        mesh=plsc.VectorSubcoreMesh(core_axis_name="c", subcore_axis_name="s"),
        scratch_types=[
            pltpu.VMEM((_G_CH,), jnp.int32),
            pltpu.VMEM((_G_CH, D), jnp.float32),
            pltpu.SemaphoreType.DMA,
        ],
    )(_dispatch_body)


def _dispatch_body(x_hbm, idx_hbm, out_hbm, idx_v, rows_v, sem):
    wid = lax.axis_index("s") * NC + lax.axis_index("c")
    base = wid * _G_PER_W

    def chunk(i, carry):
        off = base + i * _G_CH
        pltpu.sync_copy(idx_hbm.at[pl.ds(off, _G_CH)], idx_v)
        pltpu.async_copy(x_hbm.at[idx_v], rows_v, sem).wait()
        pltpu.sync_copy(rows_v, out_hbm.at[pl.ds(off, _G_CH)])
        return carry

    lax.fori_loop(0, _G_PER_W // _G_CH, chunk, 0)


def _ffn_body(be_ref, x_ref, w1_ref, b1_ref, w2_ref, b2_ref, ws_ref, o_ref):
    xb = x_ref[...].astype(jnp.bfloat16)
    h = jnp.dot(xb, w1_ref[0], preferred_element_type=jnp.float32)
    h = h + b1_ref[0]
    h = h / (1.0 + jnp.exp(-h))
    y = jnp.dot(h.astype(jnp.bfloat16), w2_ref[0],
                preferred_element_type=jnp.float32)
    o_ref[...] = (y + b2_ref[0]) * ws_ref[...]


def _grouped_ffn(block_expert, x_sorted, w1, b1, w2, b2, w_sorted):
    grid_spec = pltpu.PrefetchScalarGridSpec(
        num_scalar_prefetch=1,
        grid=(NB,),
        in_specs=[
            pl.BlockSpec((BLK, D), lambda b, be: (b, 0)),
            pl.BlockSpec((1, D, FF), lambda b, be: (be[b], 0, 0)),
            pl.BlockSpec((1, 1, FF), lambda b, be: (be[b], 0, 0)),
            pl.BlockSpec((1, FF, D), lambda b, be: (be[b], 0, 0)),
            pl.BlockSpec((1, 1, D), lambda b, be: (be[b], 0, 0)),
            pl.BlockSpec((BLK, 1), lambda b, be: (b, 0)),
        ],
        out_specs=pl.BlockSpec((BLK, D), lambda b, be: (b, 0)),
    )
    return pl.pallas_call(
        _ffn_body,
        grid_spec=grid_spec,
        out_shape=jax.ShapeDtypeStruct((P, D), jnp.float32),
    )(block_expert, x_sorted, w1, b1, w2, b2, w_sorted)


_C_CH = 16
_C_PER_W = T // NW


@functools.cache
def _make_combine():
    return functools.partial(
        pl.kernel,
        out_type=jax.ShapeDtypeStruct((T, D), jnp.float32),
        mesh=plsc.VectorSubcoreMesh(core_axis_name="c", subcore_axis_name="s"),
        scratch_types=[
            pltpu.VMEM((_C_CH,), jnp.int32),
            pltpu.VMEM((_C_CH,), jnp.int32),
            pltpu.VMEM((_C_CH, D), jnp.float32),
            pltpu.VMEM((_C_CH, D), jnp.float32),
            pltpu.SemaphoreType.DMA,
            pltpu.SemaphoreType.DMA,
        ],
    )(_combine_body)


def _combine_body(y_hbm, d0_hbm, d1_hbm, out_hbm,
                  i0_v, i1_v, r0_v, r1_v, sem0, sem1):
    wid = lax.axis_index("s") * NC + lax.axis_index("c")
    base = wid * _C_PER_W

    def chunk(i, carry):
        off = base + i * _C_CH
        pltpu.sync_copy(d0_hbm.at[pl.ds(off, _C_CH)], i0_v)
        pltpu.sync_copy(d1_hbm.at[pl.ds(off, _C_CH)], i1_v)
        c0 = pltpu.async_copy(y_hbm.at[i0_v], r0_v, sem0)
        c1 = pltpu.async_copy(y_hbm.at[i1_v], r1_v, sem1)
        c0.wait()
        c1.wait()
        for r in range(_C_CH):
            def col(c, carry2):
                sl = pl.ds(c * 16, 16)
                r0_v[r, sl] = r0_v[r, sl] + r1_v[r, sl]
                return carry2

            lax.fori_loop(0, D // 16, col, 0)
        pltpu.sync_copy(r0_v, out_hbm.at[pl.ds(off, _C_CH)])
        return carry

    lax.fori_loop(0, _C_PER_W // _C_CH, chunk, 0)


def kernel(inputs, gate_w, gate_b, w1, b1, w2, b2):
    a1, a2, p1, p2 = _router(inputs, gate_w, gate_b)
    src_token, w_sorted, block_expert, d0, d1 = _route_metadata(a1, a2, p1, p2)
    x_sorted = _make_dispatch()(inputs, src_token)
    y_sorted = _grouped_ffn(block_expert, x_sorted,
                            w1.astype(jnp.bfloat16), b1.reshape(E, 1, FF),
                            w2.astype(jnp.bfloat16), b2.reshape(E, 1, D),
                            w_sorted)
    return _make_combine()(y_sorted, d0, d1)
